# baseline (speedup 1.0000x reference)
import jax
import jax.numpy as jnp
from jax.experimental import pallas as pl
from jax.experimental.pallas import tpu as pltpu

_MAX_SIZE = 50
_HALF = 128
_BATCH = 64


def _pos_broadcast_kernel(row_ref, col_ref, out_ref):
    row = row_ref[...]
    col = col_ref[...]
    out_ref[0, :, :, :_HALF] = jnp.broadcast_to(
        row[:, None, :], (_MAX_SIZE, _MAX_SIZE, _HALF)
    )
    out_ref[0, :, :, _HALF:] = jnp.broadcast_to(
        col[None, :, :], (_MAX_SIZE, _MAX_SIZE, _HALF)
    )


def kernel(batch_size, row_embed, col_embed):
    zero = (jnp.asarray(batch_size) - _BATCH).astype(row_embed.dtype)
    row_embed = row_embed + zero
    col_embed = col_embed + zero

    out = pl.pallas_call(
        _pos_broadcast_kernel,
        grid=(_BATCH,),
        in_specs=[
            pl.BlockSpec((_MAX_SIZE, _HALF), lambda b: (0, 0)),
            pl.BlockSpec((_MAX_SIZE, _HALF), lambda b: (0, 0)),
        ],
        out_specs=pl.BlockSpec(
            (1, _MAX_SIZE, _MAX_SIZE, 2 * _HALF), lambda b: (b, 0, 0, 0)
        ),
        out_shape=jax.ShapeDtypeStruct(
            (_BATCH, _MAX_SIZE, _MAX_SIZE, 2 * _HALF), row_embed.dtype
        ),
    )(row_embed, col_embed)
    return out.reshape(_BATCH, _MAX_SIZE * _MAX_SIZE, 2 * _HALF)


# speedup vs baseline: 1.0068x; 1.0068x over previous
import jax
import jax.numpy as jnp
from jax.experimental import pallas as pl
from jax.experimental.pallas import tpu as pltpu

_MAX_SIZE = 50
_HALF = 128
_BATCH = 64
_NSEM = 8


def _pos_broadcast_kernel(row_ref, col_ref, out_ref, scratch, sems):
    row = row_ref[...]
    col = col_ref[...]
    scratch[:, :, :_HALF] = jnp.broadcast_to(
        row[:, None, :], (_MAX_SIZE, _MAX_SIZE, _HALF)
    )
    scratch[:, :, _HALF:] = jnp.broadcast_to(
        col[None, :, :], (_MAX_SIZE, _MAX_SIZE, _HALF)
    )

    def start(b):
        pltpu.make_async_copy(
            scratch, out_ref.at[b], sems.at[b % _NSEM]
        ).start()

    def wait(b):
        pltpu.make_async_copy(
            scratch, out_ref.at[b], sems.at[b % _NSEM]
        ).wait()

    def body(b, _):
        wait(b - _NSEM)
        start(b)
        return 0

    for b in range(_NSEM):
        start(b)
    jax.lax.fori_loop(_NSEM, _BATCH, body, 0)
    for b in range(_BATCH - _NSEM, _BATCH):
        wait(b)


def kernel(batch_size, row_embed, col_embed):
    zero = (jnp.asarray(batch_size) - _BATCH).astype(row_embed.dtype)
    row_embed = row_embed + zero
    col_embed = col_embed + zero

    out = pl.pallas_call(
        _pos_broadcast_kernel,
        in_specs=[
            pl.BlockSpec(memory_space=pltpu.MemorySpace.VMEM),
            pl.BlockSpec(memory_space=pltpu.MemorySpace.VMEM),
        ],
        out_specs=pl.BlockSpec(memory_space=pltpu.MemorySpace.HBM),
        out_shape=jax.ShapeDtypeStruct(
            (_BATCH, _MAX_SIZE, _MAX_SIZE, 2 * _HALF), row_embed.dtype
        ),
        scratch_shapes=[
            pltpu.VMEM((_MAX_SIZE, _MAX_SIZE, 2 * _HALF), row_embed.dtype),
            pltpu.SemaphoreType.DMA((_NSEM,)),
        ],
    )(row_embed, col_embed)
    return out.reshape(_BATCH, _MAX_SIZE * _MAX_SIZE, 2 * _HALF)
